# Initial kernel scaffold; baseline (speedup 1.0000x reference)
import jax
import jax.numpy as jnp
from jax.experimental import pallas as pl


def kernel(x, edge_index, batch, W_in, b_in, W1s, b1s, W2s, b2s, eps, Wh1, bh1, Wh2, bh2):
    raise NotImplementedError("write your pallas kernel here")


# speedup vs baseline: 4.4742x; 4.4742x over previous
import functools

import jax
import jax.numpy as jnp
from jax import lax
from jax.experimental import pallas as pl
from jax.experimental.pallas import tpu as pltpu
from jax.experimental.pallas import tpu_sc as plsc

N_NODES = 10000
N_EDGES = 320000
D = 128
N_GRAPHS = 128
N_CLASSES = 6
N_LAYERS = 5

NSC = 2
NTILES = 16
NWKR = NSC * NTILES
EDGES_PER_TILE = N_EDGES // NWKR
EDGE_WIN = 80
N_WIN = EDGES_PER_TILE // EDGE_WIN
N_PAD = 10240
ROWS_PER_TILE = N_PAD // NTILES
ZERO_ROWS = 128

ROW_BLK = 1000
N_BLKS = N_NODES // ROW_BLK


def _sc_edge_scatter(h, src, dst):
    mesh = plsc.VectorSubcoreMesh(core_axis_name="c", subcore_axis_name="s")

    @functools.partial(
        pl.kernel,
        out_type=jax.ShapeDtypeStruct((NSC * N_PAD, D), jnp.float32),
        mesh=mesh,
        scratch_types=[
            pltpu.VMEM((EDGE_WIN,), jnp.int32),
            pltpu.VMEM((EDGE_WIN,), jnp.int32),
            pltpu.VMEM((EDGE_WIN, D), jnp.float32),
            pltpu.VMEM((ZERO_ROWS, D), jnp.float32),
            pltpu.VMEM_SHARED((N_PAD, D), jnp.float32),
            pltpu.SemaphoreType.DMA,
        ],
    )
    def scatter_kernel(h_hbm, src_hbm, dst_hbm, out_hbm,
                       sidx, didx, rows, zbuf, agg_sh, sem):
        cid = lax.axis_index("c")
        sid = lax.axis_index("s")
        wid = cid * NTILES + sid
        zv = jnp.zeros((16,), jnp.float32)

        @pl.loop(0, ZERO_ROWS)
        def _(r):
            for c in range(0, D, 16):
                zbuf[r, pl.ds(c, 16)] = zv

        @pl.loop(0, ROWS_PER_TILE // ZERO_ROWS)
        def _(j):
            pltpu.sync_copy(
                zbuf,
                agg_sh.at[pl.ds(sid * ROWS_PER_TILE + j * ZERO_ROWS, ZERO_ROWS)])

        plsc.subcore_barrier()

        base = wid * EDGES_PER_TILE

        @pl.loop(0, N_WIN)
        def _(w):
            e0 = base + w * EDGE_WIN
            pltpu.sync_copy(src_hbm.at[pl.ds(e0, EDGE_WIN)], sidx)
            pltpu.sync_copy(dst_hbm.at[pl.ds(e0, EDGE_WIN)], didx)
            pltpu.async_copy(h_hbm.at[sidx], rows, sem).wait()
            pltpu.sync_copy(rows, agg_sh.at[didx], add=True)

        plsc.subcore_barrier()
        r0 = sid * ROWS_PER_TILE
        pltpu.sync_copy(
            agg_sh.at[pl.ds(r0, ROWS_PER_TILE)],
            out_hbm.at[pl.ds(cid * N_PAD + r0, ROWS_PER_TILE)])

    return scatter_kernel(h, src, dst)


def _encoder_body(x_ref, w_ref, b_ref, o_ref):
    z = jnp.dot(x_ref[...], w_ref[...], preferred_element_type=jnp.float32)
    o_ref[...] = jnp.maximum(z + b_ref[...], 0.0)


def _tc_encoder(x, W, b):
    return pl.pallas_call(
        _encoder_body,
        grid=(N_BLKS,),
        in_specs=[
            pl.BlockSpec((ROW_BLK, D), lambda i: (i, 0)),
            pl.BlockSpec((D, D), lambda i: (0, 0)),
            pl.BlockSpec((1, D), lambda i: (0, 0)),
        ],
        out_specs=pl.BlockSpec((ROW_BLK, D), lambda i: (i, 0)),
        out_shape=jax.ShapeDtypeStruct((N_NODES, D), jnp.float32),
    )(x, W, b)


def _gin_body(scale_ref, h_ref, agg_ref, w1_ref, b1_ref, w2_ref, b2_ref, o_ref):
    z = scale_ref[...] * h_ref[...] + agg_ref[0] + agg_ref[1]
    z = jnp.maximum(
        jnp.dot(z, w1_ref[...], preferred_element_type=jnp.float32) + b1_ref[...],
        0.0)
    z = jnp.dot(z, w2_ref[...], preferred_element_type=jnp.float32) + b2_ref[...]
    o_ref[...] = jnp.maximum(z, 0.0)


def _tc_gin_layer(h, agg2, scale, W1, b1, W2, b2):
    return pl.pallas_call(
        _gin_body,
        grid=(N_BLKS,),
        in_specs=[
            pl.BlockSpec((1, D), lambda i: (0, 0)),
            pl.BlockSpec((ROW_BLK, D), lambda i: (i, 0)),
            pl.BlockSpec((NSC, ROW_BLK, D), lambda i: (0, i, 0)),
            pl.BlockSpec((D, D), lambda i: (0, 0)),
            pl.BlockSpec((1, D), lambda i: (0, 0)),
            pl.BlockSpec((D, D), lambda i: (0, 0)),
            pl.BlockSpec((1, D), lambda i: (0, 0)),
        ],
        out_specs=pl.BlockSpec((ROW_BLK, D), lambda i: (i, 0)),
        out_shape=jax.ShapeDtypeStruct((N_NODES, D), jnp.float32),
    )(scale, h, agg2, W1, b1, W2, b2)


def _pool_head_body(h_ref, b_ref, wh1_ref, bh1_ref, wh2_ref, bh2_ref,
                    o_ref, sums, counts):
    i = pl.program_id(0)

    @pl.when(i == 0)
    def _():
        sums[...] = jnp.zeros_like(sums)
        counts[...] = jnp.zeros_like(counts)

    gid = lax.broadcasted_iota(jnp.int32, (ROW_BLK, N_GRAPHS), 1)
    onehot = (b_ref[...] == gid).astype(jnp.float32)
    dn = (((0,), (0,)), ((), ()))
    sums[...] += lax.dot_general(onehot, h_ref[...], dn,
                                 preferred_element_type=jnp.float32)
    counts[...] += lax.dot_general(onehot, jnp.ones((ROW_BLK, D), jnp.float32),
                                   dn, preferred_element_type=jnp.float32)

    @pl.when(i == N_BLKS - 1)
    def _():
        g = sums[...] / jnp.maximum(counts[...], 1.0)
        t = jnp.maximum(
            jnp.dot(g, wh1_ref[...], preferred_element_type=jnp.float32)
            + bh1_ref[...], 0.0)
        o_ref[...] = (jnp.dot(t, wh2_ref[...], preferred_element_type=jnp.float32)
                      + bh2_ref[...])


def _tc_pool_head(h, batch_b, Wh1, bh1, Wh2p, bh2p):
    return pl.pallas_call(
        _pool_head_body,
        grid=(N_BLKS,),
        in_specs=[
            pl.BlockSpec((ROW_BLK, D), lambda i: (i, 0)),
            pl.BlockSpec((ROW_BLK, N_GRAPHS), lambda i: (i, 0)),
            pl.BlockSpec((D, D), lambda i: (0, 0)),
            pl.BlockSpec((1, D), lambda i: (0, 0)),
            pl.BlockSpec((D, D), lambda i: (0, 0)),
            pl.BlockSpec((1, D), lambda i: (0, 0)),
        ],
        out_specs=pl.BlockSpec((N_GRAPHS, D), lambda i: (0, 0)),
        out_shape=jax.ShapeDtypeStruct((N_GRAPHS, D), jnp.float32),
        scratch_shapes=[
            pltpu.VMEM((N_GRAPHS, D), jnp.float32),
            pltpu.VMEM((N_GRAPHS, D), jnp.float32),
        ],
    )(h, batch_b, Wh1, bh1, Wh2p, bh2p)


def kernel(x, edge_index, batch, W_in, b_in, W1s, b1s, W2s, b2s, eps,
           Wh1, bh1, Wh2, bh2):
    src = edge_index[0]
    dst = edge_index[1]
    h = _tc_encoder(x, W_in, b_in.reshape(1, D))
    for l in range(N_LAYERS):
        aggf = _sc_edge_scatter(h, src, dst).reshape(NSC, N_PAD, D)
        scale = (1.0 + eps[l]) * jnp.ones((1, D), jnp.float32)
        h = _tc_gin_layer(h, aggf, scale, W1s[l], b1s[l].reshape(1, D),
                          W2s[l], b2s[l].reshape(1, D))
    batch_b = jnp.broadcast_to(batch[:, None], (N_NODES, N_GRAPHS))
    Wh2p = jnp.zeros((D, D), jnp.float32).at[:, :N_CLASSES].set(Wh2)
    bh2p = jnp.zeros((1, D), jnp.float32).at[0, :N_CLASSES].set(bh2)
    out = _tc_pool_head(h, batch_b, Wh1, bh1.reshape(1, D), Wh2p, bh2p)
    return out[:, :N_CLASSES]
